# baseline (speedup 1.0000x reference)
import functools
import math

import jax
import jax.numpy as jnp
from jax import lax
from jax.experimental import pallas as pl
from jax.experimental.pallas import tpu as pltpu
from jax.experimental.pallas import tpu_sc as plsc

VOCAB = 100000
D = 512
BATCH = 16
SEQ = 2048
L = 16
NC = 2
NS = 16
NW = NC * NS
N = BATCH * SEQ
B_PER_W = N // NW
C = 64
N_CHUNKS = B_PER_W // C
SCALE = math.sqrt(float(D))


def _pe_table():
    pos = jnp.arange(SEQ, dtype=jnp.float32).reshape(-1, 1)
    i = jnp.arange(D, dtype=jnp.float32)
    rads = pos / jnp.power(10000.0, 2.0 * jnp.floor(i / 2.0) / D)
    pe = jnp.zeros((SEQ, D), dtype=jnp.float32)
    pe = pe.at[:, 0::2].set(jnp.sin(rads[:, 0::2]))
    pe = pe.at[:, 1::2].set(jnp.cos(rads[:, 1::2]))
    return pe


@functools.partial(
    pl.kernel,
    out_type=jax.ShapeDtypeStruct((N, D), jnp.float32),
    mesh=plsc.VectorSubcoreMesh(core_axis_name="c", subcore_axis_name="s"),
    scratch_types=[
        pltpu.VMEM((C,), jnp.int32),
        pltpu.VMEM((C, D), jnp.float32),
        pltpu.VMEM((C, D), jnp.float32),
        pltpu.SemaphoreType.DMA,
    ],
)
def _emb_lookup(table_hbm, idx_hbm, pe_hbm, out_hbm, idx_v, pe_v, rows_v, sem):
    wid = lax.axis_index("s") * NC + lax.axis_index("c")
    row_base = wid * B_PER_W
    pos_base = lax.rem(row_base, SEQ)

    def chunk_body(i, carry):
        base = row_base + i * C
        pltpu.sync_copy(idx_hbm.at[pl.ds(base, C)], idx_v)
        pltpu.sync_copy(pe_hbm.at[pl.ds(pos_base + i * C, C)], pe_v)
        pltpu.async_copy(table_hbm.at[idx_v], rows_v, sem).wait()

        def row_body(r, rcarry):
            for c in range(D // L):
                sl = pl.ds(c * L, L)
                rows_v[r, sl] = rows_v[r, sl] * SCALE + pe_v[r, sl]
            return rcarry

        lax.fori_loop(0, C, row_body, 0)
        pltpu.sync_copy(rows_v, out_hbm.at[pl.ds(base, C)])
        return carry

    lax.fori_loop(0, N_CHUNKS, chunk_body, 0)


def kernel(x, table):
    idx = x.reshape(N)
    pe = _pe_table()
    out = _emb_lookup(table, idx, pe)
    return out.reshape(BATCH, SEQ, D)


# speedup vs baseline: 1.3907x; 1.3907x over previous
import functools
import math

import jax
import jax.numpy as jnp
from jax import lax
from jax.experimental import pallas as pl
from jax.experimental.pallas import tpu as pltpu
from jax.experimental.pallas import tpu_sc as plsc

VOCAB = 100000
D = 512
BATCH = 16
SEQ = 2048
L = 16
NC = 2
NS = 16
NW = NC * NS
N = BATCH * SEQ
B_PER_W = N // NW
C = 32
N_CHUNKS = B_PER_W // C
SCALE = math.sqrt(float(D))


def _pe_table():
    pos = jnp.arange(SEQ, dtype=jnp.float32).reshape(-1, 1)
    i = jnp.arange(D, dtype=jnp.float32)
    rads = pos / jnp.power(10000.0, 2.0 * jnp.floor(i / 2.0) / D)
    pe = jnp.zeros((SEQ, D), dtype=jnp.float32)
    pe = pe.at[:, 0::2].set(jnp.sin(rads[:, 0::2]))
    pe = pe.at[:, 1::2].set(jnp.cos(rads[:, 1::2]))
    return pe


@functools.partial(
    pl.kernel,
    out_type=jax.ShapeDtypeStruct((N, D), jnp.float32),
    mesh=plsc.VectorSubcoreMesh(core_axis_name="c", subcore_axis_name="s"),
    scratch_types=[
        pltpu.VMEM((C,), jnp.int32),
        pltpu.VMEM((C,), jnp.int32),
        pltpu.VMEM((C, D), jnp.float32),
        pltpu.VMEM((C, D), jnp.float32),
        pltpu.VMEM((C, D), jnp.float32),
        pltpu.VMEM((C, D), jnp.float32),
        pltpu.VMEM((C, D), jnp.float32),
        pltpu.VMEM((C, D), jnp.float32),
        pltpu.SemaphoreType.DMA,
        pltpu.SemaphoreType.DMA,
        pltpu.SemaphoreType.DMA,
        pltpu.SemaphoreType.DMA,
        pltpu.SemaphoreType.DMA,
        pltpu.SemaphoreType.DMA,
    ],
)
def _emb_lookup(table_hbm, idx_hbm, pe_hbm, out_hbm,
                idx_v0, idx_v1, rows0, rows1, pe0, pe1, ost0, ost1,
                g0, g1, p0, p1, o0, o1):
    idx_v = (idx_v0, idx_v1)
    rows = (rows0, rows1)
    peb = (pe0, pe1)
    ost = (ost0, ost1)
    gs = (g0, g1)
    ps = (p0, p1)
    osm = (o0, o1)

    wid = lax.axis_index("s") * NC + lax.axis_index("c")
    row_base = wid * B_PER_W
    pos_base = lax.rem(row_base, SEQ)

    def idx_slice(j):
        return idx_hbm.at[pl.ds(row_base + j * C, C)]

    def pe_slice(j):
        return pe_hbm.at[pl.ds(pos_base + j * C, C)]

    def out_slice(j):
        return out_hbm.at[pl.ds(row_base + j * C, C)]

    pltpu.sync_copy(idx_slice(0), idx_v[0])
    pltpu.async_copy(table_hbm.at[idx_v[0]], rows[0], gs[0])
    pltpu.async_copy(pe_slice(0), peb[0], ps[0])

    @pl.loop(0, N_CHUNKS, step=2)
    def _chunk_pair(j):
        for b in (0, 1):
            chunk = j + b
            nb = 1 - b

            @pl.when(chunk + 1 < N_CHUNKS)
            def _():
                pltpu.sync_copy(idx_slice(chunk + 1), idx_v[nb])
                pltpu.async_copy(table_hbm.at[idx_v[nb]], rows[nb], gs[nb])
                pltpu.async_copy(pe_slice(chunk + 1), peb[nb], ps[nb])

            pltpu.make_async_copy(table_hbm.at[idx_v[b]], rows[b], gs[b]).wait()
            pltpu.make_async_copy(pe_slice(chunk), peb[b], ps[b]).wait()

            @pl.when(chunk >= 2)
            def _():
                pltpu.make_async_copy(ost[b], out_slice(chunk - 2), osm[b]).wait()

            def row_body(r, rc):
                for c in range(D // L):
                    sl = pl.ds(c * L, L)
                    ost[b][r, sl] = rows[b][r, sl] * SCALE + peb[b][r, sl]
                return rc

            lax.fori_loop(0, C, row_body, 0)
            pltpu.async_copy(ost[b], out_slice(chunk), osm[b])

    pltpu.make_async_copy(ost[0], out_slice(N_CHUNKS - 2), osm[0]).wait()
    pltpu.make_async_copy(ost[1], out_slice(N_CHUNKS - 1), osm[1]).wait()


def kernel(x, table):
    idx = x.reshape(N)
    pe = _pe_table()
    out = _emb_lookup(table, idx, pe)
    return out.reshape(BATCH, SEQ, D)


# speedup vs baseline: 1.6301x; 1.1721x over previous
import functools
import math

import jax
import jax.numpy as jnp
from jax import lax
from jax.experimental import pallas as pl
from jax.experimental.pallas import tpu as pltpu
from jax.experimental.pallas import tpu_sc as plsc

VOCAB = 100000
D = 512
BATCH = 16
SEQ = 2048
L = 16
NC = 2
NS = 16
NW = NC * NS
N = BATCH * SEQ
P_PER_W = SEQ // NW
C = 32
N_CHUNKS = BATCH * P_PER_W // C
SCALE = math.sqrt(float(D))


def _pe_table():
    pos = jnp.arange(SEQ, dtype=jnp.float32).reshape(-1, 1)
    i = jnp.arange(D, dtype=jnp.float32)
    rads = pos / jnp.power(10000.0, 2.0 * jnp.floor(i / 2.0) / D)
    pe = jnp.zeros((SEQ, D), dtype=jnp.float32)
    pe = pe.at[:, 0::2].set(jnp.sin(rads[:, 0::2]))
    pe = pe.at[:, 1::2].set(jnp.cos(rads[:, 1::2]))
    return pe


@functools.partial(
    pl.kernel,
    out_type=jax.ShapeDtypeStruct((N, D), jnp.float32),
    mesh=plsc.VectorSubcoreMesh(core_axis_name="c", subcore_axis_name="s"),
    scratch_types=[
        pltpu.VMEM((BATCH, 2 * P_PER_W), jnp.int32),
        pltpu.VMEM((P_PER_W, D), jnp.float32),
        pltpu.VMEM((C, D), jnp.float32),
        pltpu.VMEM((C, D), jnp.float32),
        pltpu.VMEM((C, D), jnp.float32),
        pltpu.VMEM((C, D), jnp.float32),
        pltpu.SemaphoreType.DMA,
        pltpu.SemaphoreType.DMA,
        pltpu.SemaphoreType.DMA,
        pltpu.SemaphoreType.DMA,
    ],
)
def _emb_lookup(table_hbm, idx_hbm, pe_hbm, out_hbm,
                idx_all, pe_v, rows0, rows1, ost0, ost1,
                g0, g1, o0, o1):
    rows = (rows0, rows1)
    ost = (ost0, ost1)
    gsm = (g0, g1)
    osm = (o0, o1)

    wid = lax.axis_index("s") * NC + lax.axis_index("c")
    pos0 = wid * P_PER_W

    half = lax.rem(wid, 2) * P_PER_W
    pltpu.sync_copy(pe_hbm.at[pl.ds(pos0, P_PER_W)], pe_v)
    pltpu.sync_copy(idx_hbm.at[:, pl.ds((wid // 2) * (2 * P_PER_W),
                                        2 * P_PER_W)], idx_all)

    def idx_slice(bj, h):
        return idx_all.at[bj, pl.ds(half + h * C, C)]

    def out_slice(bj, h):
        return out_hbm.at[pl.ds(bj * SEQ + pos0 + h * C, C)]

    pltpu.async_copy(table_hbm.at[idx_slice(0, 0)], rows[0], gsm[0])

    @pl.loop(0, BATCH)
    def _batch(bj):
        for h in (0, 1):
            j = 2 * bj + h

            if h == 0:
                pltpu.async_copy(table_hbm.at[idx_slice(bj, 1)],
                                 rows[1], gsm[1])
            else:
                @pl.when(bj + 1 < BATCH)
                def _():
                    pltpu.async_copy(table_hbm.at[idx_slice(bj + 1, 0)],
                                     rows[0], gsm[0])

            pltpu.make_async_copy(table_hbm.at[idx_slice(bj, h)],
                                  rows[h], gsm[h]).wait()

            @pl.when(j >= 2)
            def _():
                pltpu.make_async_copy(ost[h], out_slice(bj - 1, h),
                                      osm[h]).wait()

            def row_body(r, rc, _h=h):
                for c in range(D // L):
                    sl = pl.ds(c * L, L)
                    ost[_h][r, sl] = (rows[_h][r, sl] * SCALE
                                      + pe_v[_h * C + r, sl])
                return rc

            lax.fori_loop(0, C, row_body, 0)
            pltpu.async_copy(ost[h], out_slice(bj, h), osm[h])

    pltpu.make_async_copy(ost[0], out_slice(BATCH - 1, 0), osm[0]).wait()
    pltpu.make_async_copy(ost[1], out_slice(BATCH - 1, 1), osm[1]).wait()


def kernel(x, table):
    pe = _pe_table()
    out = _emb_lookup(table, x, pe)
    return out.reshape(BATCH, SEQ, D)


# speedup vs baseline: 2.2005x; 1.3499x over previous
import functools
import math

import jax
import jax.numpy as jnp
import numpy as np
from jax import lax
from jax.experimental import pallas as pl
from jax.experimental.pallas import tpu as pltpu
from jax.experimental.pallas import tpu_sc as plsc

VOCAB = 100000
D = 512
BATCH = 16
SEQ = 2048
L = 16
NC = 2
NS = 16
NW = NC * NS
N = BATCH * SEQ
P_PER_W = SEQ // NW
C = 32
N_CHUNKS = BATCH * P_PER_W // C
SCALE = math.sqrt(float(D))


def _pe_table():
    pos = np.arange(SEQ, dtype=np.float32).reshape(-1, 1)
    i = np.arange(D, dtype=np.float32)
    rads = pos / np.power(10000.0, 2.0 * np.floor(i / 2.0) / D)
    pe = np.zeros((SEQ, D), dtype=np.float32)
    pe[:, 0::2] = np.sin(rads[:, 0::2])
    pe[:, 1::2] = np.cos(rads[:, 1::2])
    return pe


_PE = _pe_table()


@functools.partial(
    pl.kernel,
    out_type=jax.ShapeDtypeStruct((N, D), jnp.float32),
    mesh=plsc.VectorSubcoreMesh(core_axis_name="c", subcore_axis_name="s"),
    scratch_types=[
        pltpu.VMEM((BATCH, 2 * P_PER_W), jnp.int32),
        pltpu.VMEM((P_PER_W, D), jnp.float32),
        pltpu.VMEM((C, D), jnp.float32),
        pltpu.VMEM((C, D), jnp.float32),
        pltpu.VMEM((C, D), jnp.float32),
        pltpu.VMEM((C, D), jnp.float32),
        pltpu.SemaphoreType.DMA,
        pltpu.SemaphoreType.DMA,
        pltpu.SemaphoreType.DMA,
        pltpu.SemaphoreType.DMA,
    ],
)
def _emb_lookup(table_hbm, idx_hbm, pe_hbm, out_hbm,
                idx_all, pe_v, rows0, rows1, ost0, ost1,
                g0, g1, o0, o1):
    rows = (rows0, rows1)
    ost = (ost0, ost1)
    gsm = (g0, g1)
    osm = (o0, o1)

    wid = lax.axis_index("s") * NC + lax.axis_index("c")
    pos0 = wid * P_PER_W

    half = lax.rem(wid, 2) * P_PER_W
    pltpu.sync_copy(pe_hbm.at[pl.ds(pos0, P_PER_W)], pe_v)
    pltpu.sync_copy(idx_hbm.at[:, pl.ds((wid // 2) * (2 * P_PER_W),
                                        2 * P_PER_W)], idx_all)

    def idx_slice(bj, h):
        return idx_all.at[bj, pl.ds(half + h * C, C)]

    def out_slice(bj, h):
        return out_hbm.at[pl.ds(bj * SEQ + pos0 + h * C, C)]

    pltpu.async_copy(table_hbm.at[idx_slice(0, 0)], rows[0], gsm[0])

    @pl.loop(0, BATCH)
    def _batch(bj):
        for h in (0, 1):
            j = 2 * bj + h

            if h == 0:
                pltpu.async_copy(table_hbm.at[idx_slice(bj, 1)],
                                 rows[1], gsm[1])
            else:
                @pl.when(bj + 1 < BATCH)
                def _():
                    pltpu.async_copy(table_hbm.at[idx_slice(bj + 1, 0)],
                                     rows[0], gsm[0])

            pltpu.make_async_copy(table_hbm.at[idx_slice(bj, h)],
                                  rows[h], gsm[h]).wait()

            @pl.when(j >= 2)
            def _():
                pltpu.make_async_copy(ost[h], out_slice(bj - 1, h),
                                      osm[h]).wait()

            def row_body(r, rc, _h=h):
                for c in range(D // L):
                    sl = pl.ds(c * L, L)
                    ost[_h][r, sl] = (rows[_h][r, sl] * SCALE
                                      + pe_v[_h * C + r, sl])
                return rc

            lax.fori_loop(0, C, row_body, 0)
            pltpu.async_copy(ost[h], out_slice(bj, h), osm[h])

    pltpu.make_async_copy(ost[0], out_slice(BATCH - 1, 0), osm[0]).wait()
    pltpu.make_async_copy(ost[1], out_slice(BATCH - 1, 1), osm[1]).wait()


def kernel(x, table):
    pe = jnp.asarray(_PE)
    out = _emb_lookup(table, x, pe)
    return out.reshape(BATCH, SEQ, D)
